# baseline (speedup 1.0000x reference)
import functools

import jax
import jax.numpy as jnp
import numpy as np
from jax import lax
from jax.experimental import pallas as pl
from jax.experimental.pallas import tpu as pltpu
from jax.experimental.pallas import tpu_sc as plsc

_EPS = 1e-5
_NPB = 800
_NPB2 = 2000


def _sc_gather(table, idx):
    V, D = table.shape
    E = idx.shape[0]
    info = plsc.get_sparse_core_info()
    NC, NS = info.num_cores, info.num_subcores
    NW = NC * NS
    assert E % NW == 0
    e_per_w = E // NW
    CH = 2000
    assert e_per_w % CH == 0
    n_ch = e_per_w // CH
    mesh = plsc.VectorSubcoreMesh(core_axis_name="c", subcore_axis_name="s")

    @functools.partial(
        pl.kernel,
        mesh=mesh,
        out_type=jax.ShapeDtypeStruct((E, D), jnp.float32),
        compiler_params=pltpu.CompilerParams(use_tc_tiling_on_sc=False),
        scratch_types=[
            pltpu.VMEM((CH,), jnp.int32),
            pltpu.VMEM((CH, D), jnp.float32),
            pltpu.SemaphoreType.DMA,
        ],
    )
    def k(table_hbm, idx_hbm, out_hbm, idx_v, rows_v, sem):
        wid = lax.axis_index("s") * NC + lax.axis_index("c")
        base = wid * e_per_w

        def body(i, carry):
            off = base + i * CH
            pltpu.sync_copy(idx_hbm.at[pl.ds(off, CH)], idx_v)
            pltpu.async_copy(table_hbm.at[idx_v], rows_v, sem).wait()
            pltpu.sync_copy(rows_v, out_hbm.at[pl.ds(off, CH)])
            return carry

        lax.fori_loop(0, n_ch, body, 0)

    return k(table, idx)


def _tc_init(na, wnrow, binrow):
    BN = na.shape[0]
    HA = wnrow.shape[1]
    grid = BN // _NPB2

    def body(na_ref, w_ref, b_ref, atom_ref):
        atom_ref[...] = na_ref[...] * w_ref[...] + b_ref[...]

    return pl.pallas_call(
        body,
        grid=(grid,),
        in_specs=[
            pl.BlockSpec((_NPB2, 1), lambda i: (i, 0)),
            pl.BlockSpec((1, HA), lambda i: (0, 0)),
            pl.BlockSpec((1, HA), lambda i: (0, 0)),
        ],
        out_specs=pl.BlockSpec((_NPB2, HA), lambda i: (i, 0)),
        out_shape=jax.ShapeDtypeStruct((BN, HA), jnp.float32),
    )(na, wnrow, binrow)


def _tc_layer(anbr3, ef3, atom, wsF, wsC, bfF, bfC,
              wnFk, wnCk, weF0, weF1, weC0, weC1,
              ghFt, bhFt, ghCt, bhCt, go2, bo2, Kfold, Tfold, S,
              final, extras):
    BN, HA = atom.shape
    grid = BN // _NPB
    Sn = float(BN)

    c0 = lambda k, i: (0, 0)
    edge_map = lambda k, i: (0, jnp.where(k == 0, i, 0), 0)
    node_map = lambda k, i: (i, 0)
    out_map = lambda k, i: (jnp.where(k == 2, i, 0), 0)

    def body(a_ref, e_ref, atom_ref, wsF_ref, wsC_ref, bfF_ref, bfC_ref,
             wnF_ref, wnC_ref, weF0_ref, weF1_ref, weC0_ref, weC1_ref,
             ghF_ref, bhF_ref, ghC_ref, bhC_ref,
             go_ref, bo_ref, K_ref, T_ref, *rest):
        if final:
            we_ref, b0_ref, out_ref, a_s, e_s, st_s, sm_s, st2_s = rest
        else:
            natom_ref, a_s, e_s, st_s, sm_s, st2_s = rest
        k = pl.program_id(0)
        i = pl.program_id(1)

        def gate_halves(get_a, get_e, wn_dtype):
            at = atom_ref[...]
            sF = (
                jnp.dot(at, wsF_ref[...], preferred_element_type=jnp.float32)
                + bfF_ref[...]
            )
            sC = (
                jnp.dot(at, wsC_ref[...], preferred_element_type=jnp.float32)
                + bfC_ref[...]
            )
            eh = [get_e(0), get_e(1)]
            weF = [weF0_ref[...], weF1_ref[...]]
            weC = [weC0_ref[...], weC1_ref[...]]
            wnF = wnF_ref[...].astype(wn_dtype)
            wnC = wnC_ref[...].astype(wn_dtype)
            outs = []
            for kk in range(4):
                a2 = get_a(kk)
                e2 = eh[kk // 2]
                gF = (
                    jnp.dot(a2, wnF, preferred_element_type=jnp.float32)
                    + jnp.dot(e2, weF[kk % 2], preferred_element_type=jnp.float32)
                    + sF
                )
                gC = (
                    jnp.dot(a2, wnC, preferred_element_type=jnp.float32)
                    + jnp.dot(e2, weC[kk % 2], preferred_element_type=jnp.float32)
                    + sC
                )
                outs.append((gF, gC))
            return outs

        @pl.when(k == 0)
        def _phase_stats():
            a_s[:, pl.ds(i * _NPB, _NPB), :] = a_ref[...].astype(jnp.bfloat16)
            e_s[:, pl.ds(i * _NPB, _NPB), :] = e_ref[...]
            s1F = s2F = s1C = s2C = None
            for gF, gC in gate_halves(
                lambda kk: a_ref[kk], lambda j: e_ref[j], jnp.float32
            ):
                p1F = jnp.sum(gF, axis=0, keepdims=True)
                p2F = jnp.sum(gF * gF, axis=0, keepdims=True)
                p1C = jnp.sum(gC, axis=0, keepdims=True)
                p2C = jnp.sum(gC * gC, axis=0, keepdims=True)
                if s1F is None:
                    s1F, s2F, s1C, s2C = p1F, p2F, p1C, p2C
                else:
                    s1F, s2F, s1C, s2C = s1F + p1F, s2F + p2F, s1C + p1C, s2C + p2C
            upd = jnp.concatenate([s1F, s2F, s1C, s2C], axis=0)

            @pl.when(i == 0)
            def _():
                st_s[...] = jnp.zeros_like(st_s)

            st_s[...] += upd

        @pl.when(k == 1)
        def _phase_apply():
            stf = jnp.dot(st_s[...], K_ref[...], preferred_element_type=jnp.float32)
            m1F = stf[0:1, :] / S
            vF = stf[1:2, :] / S - m1F * m1F
            aF = ghF_ref[...] * lax.rsqrt(vF + _EPS)
            cF = bhF_ref[...] - m1F * aF
            m1C = stf[2:3, :] / S
            vC = stf[3:4, :] / S - m1C * m1C
            aC = ghC_ref[...] * lax.rsqrt(vC + _EPS)
            cC = bhC_ref[...] - m1C * aC
            tot = None
            for gF, gC in gate_halves(
                lambda kk: a_s[kk, pl.ds(i * _NPB, _NPB), :],
                lambda j: e_s[j, pl.ds(i * _NPB, _NPB), :],
                jnp.bfloat16,
            ):
                p = jax.nn.sigmoid(gF * aF + cF) * jnp.maximum(gC * aC + cC, 0.0)
                tot = p if tot is None else tot + p
            sm = jnp.dot(tot, T_ref[...], preferred_element_type=jnp.float32)
            sm_s[pl.ds(i * _NPB, _NPB), :] = sm
            t1 = jnp.sum(sm, axis=0, keepdims=True)
            t2 = jnp.sum(sm * sm, axis=0, keepdims=True)

            @pl.when(i == 0)
            def _():
                st2_s[...] = jnp.zeros_like(st2_s)

            st2_s[...] += jnp.concatenate([t1, t2], axis=0)

        @pl.when(k == 2)
        def _phase_node():
            st_v = st2_s[...]
            m1 = st_v[0:1, :] / Sn
            v = st_v[1:2, :] / Sn - m1 * m1
            aa = go_ref[...] * lax.rsqrt(v + _EPS)
            cc = bo_ref[...] - m1 * aa
            sm = sm_s[pl.ds(i * _NPB, _NPB), :]
            na_ = jnp.maximum(atom_ref[...] + aa * sm + cc, 0.0)
            if final:
                val = jnp.sum(na_ * we_ref[...])

                @pl.when(i == 0)
                def _():
                    out_ref[...] = Sn * b0_ref[...]

                out_ref[...] += val.reshape(1, 1)
            else:
                natom_ref[...] = na_

    in_specs = [
        pl.BlockSpec((4, _NPB, 128), edge_map),
        pl.BlockSpec((2, _NPB, 128), edge_map),
        pl.BlockSpec((_NPB, HA), node_map),
        pl.BlockSpec((HA, 128), c0),
        pl.BlockSpec((HA, 128), c0),
        pl.BlockSpec((1, 128), c0),
        pl.BlockSpec((1, 128), c0),
        pl.BlockSpec((128, 128), c0),
        pl.BlockSpec((128, 128), c0),
        pl.BlockSpec((128, 128), c0),
        pl.BlockSpec((128, 128), c0),
        pl.BlockSpec((128, 128), c0),
        pl.BlockSpec((128, 128), c0),
        pl.BlockSpec((1, 128), c0),
        pl.BlockSpec((1, 128), c0),
        pl.BlockSpec((1, 128), c0),
        pl.BlockSpec((1, 128), c0),
        pl.BlockSpec((1, HA), c0),
        pl.BlockSpec((1, HA), c0),
        pl.BlockSpec((128, 128), c0),
        pl.BlockSpec((128, HA), c0),
    ]
    args = [anbr3, ef3, atom, wsF, wsC, bfF, bfC,
            wnFk, wnCk, weF0, weF1, weC0, weC1,
            ghFt, bhFt, ghCt, bhCt, go2, bo2, Kfold, Tfold]
    if final:
        werow, b0 = extras
        in_specs += [pl.BlockSpec((1, HA), c0), pl.BlockSpec((1, 1), c0)]
        args += [werow, b0]
        out_specs = pl.BlockSpec((1, 1), c0)
        out_shape = jax.ShapeDtypeStruct((1, 1), jnp.float32)
    else:
        out_specs = pl.BlockSpec((_NPB, HA), out_map)
        out_shape = jax.ShapeDtypeStruct((BN, HA), jnp.float32)

    return pl.pallas_call(
        body,
        grid=(3, grid),
        in_specs=in_specs,
        out_specs=out_specs,
        out_shape=out_shape,
        scratch_shapes=[
            pltpu.VMEM((4, BN, 128), jnp.bfloat16),
            pltpu.VMEM((2, BN, 128), jnp.bfloat16),
            pltpu.VMEM((4, 128), jnp.float32),
            pltpu.VMEM((BN, HA), jnp.float32),
            pltpu.VMEM((2, HA), jnp.float32),
        ],
        compiler_params=pltpu.CompilerParams(
            dimension_semantics=("arbitrary", "arbitrary"),
            vmem_limit_bytes=112 * 1024 * 1024,
        ),
    )(*args)


def kernel(node_attr, edge_attr, edge_idx, Wn, b_in, Wf, bf, gh, bh, go, bo, We, b_out):
    B, N, M = edge_idx.shape
    HA = Wn.shape[0]
    HB = edge_attr.shape[-1]
    BN = B * N
    E = BN * M
    MG = M // 4
    EG = M // 8

    f32 = jnp.float32
    eye4 = jnp.eye(4, dtype=f32)
    eye8 = jnp.eye(8, dtype=f32)

    na = node_attr.reshape(BN, 1)

    idx_off = edge_idx + (jnp.arange(B, dtype=edge_idx.dtype) * N)[:, None, None]
    idx_r = idx_off.reshape(BN, MG, 4).transpose(1, 0, 2).reshape(E)

    ef3 = (
        edge_attr.reshape(BN, EG, 8, HB).transpose(1, 0, 2, 3).reshape(EG, BN, 128)
    ).astype(jnp.bfloat16)

    r128 = np.arange(128)
    Kfold = jnp.asarray((r128[:, None] % HA == r128[None, :] % HA), dtype=f32)
    Tfold = jnp.asarray((r128[:, None] % HA == np.arange(HA)[None, :]), dtype=f32)

    def tile4(x):
        return jnp.concatenate([x] * 4, axis=1)

    n_layers = Wf.shape[0]
    layers = []
    for i in range(n_layers):
        Wfi = Wf[i]
        k8F = jnp.kron(eye8, Wfi[:HA, 2 * HA :].T)
        k8C = jnp.kron(eye8, Wfi[HA:, 2 * HA :].T)
        layers.append(dict(
            wsF=jnp.concatenate([Wfi[:HA, :HA].T] * 4, axis=1),
            wsC=jnp.concatenate([Wfi[HA:, :HA].T] * 4, axis=1),
            wnFk=jnp.kron(eye4, Wfi[:HA, HA : 2 * HA].T),
            wnCk=jnp.kron(eye4, Wfi[HA:, HA : 2 * HA].T),
            weF0=k8F[:, :128].astype(jnp.bfloat16),
            weF1=k8F[:, 128:].astype(jnp.bfloat16),
            weC0=k8C[:, :128].astype(jnp.bfloat16),
            weC1=k8C[:, 128:].astype(jnp.bfloat16),
            bfF=tile4(bf[i][:HA].reshape(1, HA)),
            bfC=tile4(bf[i][HA:].reshape(1, HA)),
            ghFt=tile4(gh[i][:HA].reshape(1, HA)),
            ghCt=tile4(gh[i][HA:].reshape(1, HA)),
            bhFt=tile4(bh[i][:HA].reshape(1, HA)),
            bhCt=tile4(bh[i][HA:].reshape(1, HA)),
            go2=go[i].reshape(1, HA),
            bo2=bo[i].reshape(1, HA),
        ))

    atom = _tc_init(na, Wn.reshape(1, HA), b_in.reshape(1, HA))

    out = None
    for i in range(n_layers):
        Li = layers[i]
        anbr3 = _sc_gather(atom, idx_r).reshape(MG, BN, 128)
        last = i == n_layers - 1
        extras = (We.reshape(1, HA), b_out.reshape(1, 1)) if last else None
        res = _tc_layer(
            anbr3, ef3, atom,
            Li["wsF"], Li["wsC"], Li["bfF"], Li["bfC"],
            Li["wnFk"], Li["wnCk"],
            Li["weF0"], Li["weF1"], Li["weC0"], Li["weC1"],
            Li["ghFt"], Li["bhFt"], Li["ghCt"], Li["bhCt"],
            Li["go2"], Li["bo2"], Kfold, Tfold, float(E),
            last, extras,
        )
        if last:
            out = res
        else:
            atom = res

    return out.reshape(())


# speedup vs baseline: 1.0073x; 1.0073x over previous
import functools

import jax
import jax.numpy as jnp
import numpy as np
from jax import lax
from jax.experimental import pallas as pl
from jax.experimental.pallas import tpu as pltpu
from jax.experimental.pallas import tpu_sc as plsc

_EPS = 1e-5
_NPB = 800
_NPB2 = 2000


def _sc_gather(table, idx):
    V, D = table.shape
    E = idx.shape[0]
    info = plsc.get_sparse_core_info()
    NC, NS = info.num_cores, info.num_subcores
    NW = NC * NS
    assert E % NW == 0
    e_per_w = E // NW
    CH = 1000
    assert e_per_w % CH == 0
    n_ch = e_per_w // CH
    mesh = plsc.VectorSubcoreMesh(core_axis_name="c", subcore_axis_name="s")

    @functools.partial(
        pl.kernel,
        mesh=mesh,
        out_type=jax.ShapeDtypeStruct((E, D), jnp.float32),
        compiler_params=pltpu.CompilerParams(use_tc_tiling_on_sc=False),
        scratch_types=[
            pltpu.VMEM((2, CH), jnp.int32),
            pltpu.VMEM((2, CH, D), jnp.float32),
            pltpu.SemaphoreType.DMA,
            pltpu.SemaphoreType.DMA,
        ],
    )
    def k(table_hbm, idx_hbm, out_hbm, idx_v, rows_v, gsem, osem):
        wid = lax.axis_index("s") * NC + lax.axis_index("c")
        base = wid * e_per_w

        gather_h = [None, None]
        out_h = [None, None]
        pltpu.sync_copy(idx_hbm.at[pl.ds(base, CH)], idx_v.at[0])
        gather_h[0] = pltpu.async_copy(table_hbm.at[idx_v.at[0]], rows_v.at[0], gsem)
        for j in range(n_ch):
            b = j % 2
            nb = (j + 1) % 2
            if j + 1 < n_ch:
                noff = base + (j + 1) * CH
                pltpu.sync_copy(idx_hbm.at[pl.ds(noff, CH)], idx_v.at[nb])
                if out_h[nb] is not None:
                    out_h[nb].wait()
                gather_h[nb] = pltpu.async_copy(
                    table_hbm.at[idx_v.at[nb]], rows_v.at[nb], gsem
                )
            gather_h[b].wait()
            out_h[b] = pltpu.async_copy(
                rows_v.at[b], out_hbm.at[pl.ds(base + j * CH, CH)], osem
            )
        out_h[(n_ch - 2) % 2].wait()
        out_h[(n_ch - 1) % 2].wait()

    return k(table, idx)


def _tc_init(na, wnrow, binrow):
    BN = na.shape[0]
    HA = wnrow.shape[1]
    grid = BN // _NPB2

    def body(na_ref, w_ref, b_ref, atom_ref):
        atom_ref[...] = na_ref[...] * w_ref[...] + b_ref[...]

    return pl.pallas_call(
        body,
        grid=(grid,),
        in_specs=[
            pl.BlockSpec((_NPB2, 1), lambda i: (i, 0)),
            pl.BlockSpec((1, HA), lambda i: (0, 0)),
            pl.BlockSpec((1, HA), lambda i: (0, 0)),
        ],
        out_specs=pl.BlockSpec((_NPB2, HA), lambda i: (i, 0)),
        out_shape=jax.ShapeDtypeStruct((BN, HA), jnp.float32),
    )(na, wnrow, binrow)


def _tc_layer(anbr3, ef3, atom, wsF, wsC, bfF, bfC,
              wnFk, wnCk, weF0, weF1, weC0, weC1,
              ghFt, bhFt, ghCt, bhCt, go2, bo2, Kfold, Tfold, S,
              final, extras):
    BN, HA = atom.shape
    grid = BN // _NPB
    Sn = float(BN)

    c0 = lambda k, i: (0, 0)
    edge_map = lambda k, i: (0, jnp.where(k == 0, i, 0), 0)
    node_map = lambda k, i: (i, 0)
    out_map = lambda k, i: (jnp.where(k == 2, i, 0), 0)

    def body(a_ref, e_ref, atom_ref, wsF_ref, wsC_ref, bfF_ref, bfC_ref,
             wnF_ref, wnC_ref, weF0_ref, weF1_ref, weC0_ref, weC1_ref,
             ghF_ref, bhF_ref, ghC_ref, bhC_ref,
             go_ref, bo_ref, K_ref, T_ref, *rest):
        if final:
            we_ref, b0_ref, out_ref, a_s, e_s, st_s, sm_s, st2_s = rest
        else:
            natom_ref, a_s, e_s, st_s, sm_s, st2_s = rest
        k = pl.program_id(0)
        i = pl.program_id(1)

        def gate_halves(get_a, get_e, wn_dtype):
            at = atom_ref[...]
            sF = (
                jnp.dot(at, wsF_ref[...], preferred_element_type=jnp.float32)
                + bfF_ref[...]
            )
            sC = (
                jnp.dot(at, wsC_ref[...], preferred_element_type=jnp.float32)
                + bfC_ref[...]
            )
            eh = [get_e(0), get_e(1)]
            weF = [weF0_ref[...], weF1_ref[...]]
            weC = [weC0_ref[...], weC1_ref[...]]
            wnF = wnF_ref[...].astype(wn_dtype)
            wnC = wnC_ref[...].astype(wn_dtype)
            outs = []
            for kk in range(4):
                a2 = get_a(kk)
                e2 = eh[kk // 2]
                gF = (
                    jnp.dot(a2, wnF, preferred_element_type=jnp.float32)
                    + jnp.dot(e2, weF[kk % 2], preferred_element_type=jnp.float32)
                    + sF
                )
                gC = (
                    jnp.dot(a2, wnC, preferred_element_type=jnp.float32)
                    + jnp.dot(e2, weC[kk % 2], preferred_element_type=jnp.float32)
                    + sC
                )
                outs.append((gF, gC))
            return outs

        @pl.when(k == 0)
        def _phase_stats():
            a_s[:, pl.ds(i * _NPB, _NPB), :] = a_ref[...].astype(jnp.bfloat16)
            e_s[:, pl.ds(i * _NPB, _NPB), :] = e_ref[...]
            s1F = s2F = s1C = s2C = None
            for gF, gC in gate_halves(
                lambda kk: a_ref[kk], lambda j: e_ref[j], jnp.float32
            ):
                p1F = jnp.sum(gF, axis=0, keepdims=True)
                p2F = jnp.sum(gF * gF, axis=0, keepdims=True)
                p1C = jnp.sum(gC, axis=0, keepdims=True)
                p2C = jnp.sum(gC * gC, axis=0, keepdims=True)
                if s1F is None:
                    s1F, s2F, s1C, s2C = p1F, p2F, p1C, p2C
                else:
                    s1F, s2F, s1C, s2C = s1F + p1F, s2F + p2F, s1C + p1C, s2C + p2C
            upd = jnp.concatenate([s1F, s2F, s1C, s2C], axis=0)

            @pl.when(i == 0)
            def _():
                st_s[...] = jnp.zeros_like(st_s)

            st_s[...] += upd

        @pl.when(k == 1)
        def _phase_apply():
            stf = jnp.dot(st_s[...], K_ref[...], preferred_element_type=jnp.float32)
            m1F = stf[0:1, :] / S
            vF = stf[1:2, :] / S - m1F * m1F
            aF = ghF_ref[...] * lax.rsqrt(vF + _EPS)
            cF = bhF_ref[...] - m1F * aF
            m1C = stf[2:3, :] / S
            vC = stf[3:4, :] / S - m1C * m1C
            aC = ghC_ref[...] * lax.rsqrt(vC + _EPS)
            cC = bhC_ref[...] - m1C * aC
            tot = None
            for gF, gC in gate_halves(
                lambda kk: a_s[kk, pl.ds(i * _NPB, _NPB), :],
                lambda j: e_s[j, pl.ds(i * _NPB, _NPB), :],
                jnp.bfloat16,
            ):
                p = jax.nn.sigmoid(gF * aF + cF) * jnp.maximum(gC * aC + cC, 0.0)
                tot = p if tot is None else tot + p
            sm = jnp.dot(tot, T_ref[...], preferred_element_type=jnp.float32)
            sm_s[pl.ds(i * _NPB, _NPB), :] = sm
            t1 = jnp.sum(sm, axis=0, keepdims=True)
            t2 = jnp.sum(sm * sm, axis=0, keepdims=True)

            @pl.when(i == 0)
            def _():
                st2_s[...] = jnp.zeros_like(st2_s)

            st2_s[...] += jnp.concatenate([t1, t2], axis=0)

        @pl.when(k == 2)
        def _phase_node():
            st_v = st2_s[...]
            m1 = st_v[0:1, :] / Sn
            v = st_v[1:2, :] / Sn - m1 * m1
            aa = go_ref[...] * lax.rsqrt(v + _EPS)
            cc = bo_ref[...] - m1 * aa
            sm = sm_s[pl.ds(i * _NPB, _NPB), :]
            na_ = jnp.maximum(atom_ref[...] + aa * sm + cc, 0.0)
            if final:
                val = jnp.sum(na_ * we_ref[...])

                @pl.when(i == 0)
                def _():
                    out_ref[...] = Sn * b0_ref[...]

                out_ref[...] += val.reshape(1, 1)
            else:
                natom_ref[...] = na_

    in_specs = [
        pl.BlockSpec((4, _NPB, 128), edge_map),
        pl.BlockSpec((2, _NPB, 128), edge_map),
        pl.BlockSpec((_NPB, HA), node_map),
        pl.BlockSpec((HA, 128), c0),
        pl.BlockSpec((HA, 128), c0),
        pl.BlockSpec((1, 128), c0),
        pl.BlockSpec((1, 128), c0),
        pl.BlockSpec((128, 128), c0),
        pl.BlockSpec((128, 128), c0),
        pl.BlockSpec((128, 128), c0),
        pl.BlockSpec((128, 128), c0),
        pl.BlockSpec((128, 128), c0),
        pl.BlockSpec((128, 128), c0),
        pl.BlockSpec((1, 128), c0),
        pl.BlockSpec((1, 128), c0),
        pl.BlockSpec((1, 128), c0),
        pl.BlockSpec((1, 128), c0),
        pl.BlockSpec((1, HA), c0),
        pl.BlockSpec((1, HA), c0),
        pl.BlockSpec((128, 128), c0),
        pl.BlockSpec((128, HA), c0),
    ]
    args = [anbr3, ef3, atom, wsF, wsC, bfF, bfC,
            wnFk, wnCk, weF0, weF1, weC0, weC1,
            ghFt, bhFt, ghCt, bhCt, go2, bo2, Kfold, Tfold]
    if final:
        werow, b0 = extras
        in_specs += [pl.BlockSpec((1, HA), c0), pl.BlockSpec((1, 1), c0)]
        args += [werow, b0]
        out_specs = pl.BlockSpec((1, 1), c0)
        out_shape = jax.ShapeDtypeStruct((1, 1), jnp.float32)
    else:
        out_specs = pl.BlockSpec((_NPB, HA), out_map)
        out_shape = jax.ShapeDtypeStruct((BN, HA), jnp.float32)

    return pl.pallas_call(
        body,
        grid=(3, grid),
        in_specs=in_specs,
        out_specs=out_specs,
        out_shape=out_shape,
        scratch_shapes=[
            pltpu.VMEM((4, BN, 128), jnp.bfloat16),
            pltpu.VMEM((2, BN, 128), jnp.bfloat16),
            pltpu.VMEM((4, 128), jnp.float32),
            pltpu.VMEM((BN, HA), jnp.float32),
            pltpu.VMEM((2, HA), jnp.float32),
        ],
        compiler_params=pltpu.CompilerParams(
            dimension_semantics=("arbitrary", "arbitrary"),
            vmem_limit_bytes=112 * 1024 * 1024,
        ),
    )(*args)


def kernel(node_attr, edge_attr, edge_idx, Wn, b_in, Wf, bf, gh, bh, go, bo, We, b_out):
    B, N, M = edge_idx.shape
    HA = Wn.shape[0]
    HB = edge_attr.shape[-1]
    BN = B * N
    E = BN * M
    MG = M // 4
    EG = M // 8

    f32 = jnp.float32
    eye4 = jnp.eye(4, dtype=f32)
    eye8 = jnp.eye(8, dtype=f32)

    na = node_attr.reshape(BN, 1)

    idx_off = edge_idx + (jnp.arange(B, dtype=edge_idx.dtype) * N)[:, None, None]
    idx_r = idx_off.reshape(BN, MG, 4).transpose(1, 0, 2).reshape(E)

    ef3 = (
        edge_attr.reshape(BN, EG, 8, HB).transpose(1, 0, 2, 3).reshape(EG, BN, 128)
    ).astype(jnp.bfloat16)

    r128 = np.arange(128)
    Kfold = jnp.asarray((r128[:, None] % HA == r128[None, :] % HA), dtype=f32)
    Tfold = jnp.asarray((r128[:, None] % HA == np.arange(HA)[None, :]), dtype=f32)

    def tile4(x):
        return jnp.concatenate([x] * 4, axis=1)

    n_layers = Wf.shape[0]
    layers = []
    for i in range(n_layers):
        Wfi = Wf[i]
        k8F = jnp.kron(eye8, Wfi[:HA, 2 * HA :].T)
        k8C = jnp.kron(eye8, Wfi[HA:, 2 * HA :].T)
        layers.append(dict(
            wsF=jnp.concatenate([Wfi[:HA, :HA].T] * 4, axis=1),
            wsC=jnp.concatenate([Wfi[HA:, :HA].T] * 4, axis=1),
            wnFk=jnp.kron(eye4, Wfi[:HA, HA : 2 * HA].T),
            wnCk=jnp.kron(eye4, Wfi[HA:, HA : 2 * HA].T),
            weF0=k8F[:, :128].astype(jnp.bfloat16),
            weF1=k8F[:, 128:].astype(jnp.bfloat16),
            weC0=k8C[:, :128].astype(jnp.bfloat16),
            weC1=k8C[:, 128:].astype(jnp.bfloat16),
            bfF=tile4(bf[i][:HA].reshape(1, HA)),
            bfC=tile4(bf[i][HA:].reshape(1, HA)),
            ghFt=tile4(gh[i][:HA].reshape(1, HA)),
            ghCt=tile4(gh[i][HA:].reshape(1, HA)),
            bhFt=tile4(bh[i][:HA].reshape(1, HA)),
            bhCt=tile4(bh[i][HA:].reshape(1, HA)),
            go2=go[i].reshape(1, HA),
            bo2=bo[i].reshape(1, HA),
        ))

    atom = _tc_init(na, Wn.reshape(1, HA), b_in.reshape(1, HA))

    out = None
    for i in range(n_layers):
        Li = layers[i]
        anbr3 = _sc_gather(atom, idx_r).reshape(MG, BN, 128)
        last = i == n_layers - 1
        extras = (We.reshape(1, HA), b_out.reshape(1, 1)) if last else None
        res = _tc_layer(
            anbr3, ef3, atom,
            Li["wsF"], Li["wsC"], Li["bfF"], Li["bfC"],
            Li["wnFk"], Li["wnCk"],
            Li["weF0"], Li["weF1"], Li["weC0"], Li["weC1"],
            Li["ghFt"], Li["bhFt"], Li["ghCt"], Li["bhCt"],
            Li["go2"], Li["bo2"], Kfold, Tfold, float(E),
            last, extras,
        )
        if last:
            out = res
        else:
            atom = res

    return out.reshape(())


# speedup vs baseline: 1.0790x; 1.0711x over previous
import functools

import jax
import jax.numpy as jnp
import numpy as np
from jax import lax
from jax.experimental import pallas as pl
from jax.experimental.pallas import tpu as pltpu
from jax.experimental.pallas import tpu_sc as plsc

_EPS = 1e-5
_NPB = 800
_NPB2 = 2000


def _sc_gather(table, idx):
    V, D = table.shape
    E = idx.shape[0]
    info = plsc.get_sparse_core_info()
    NC, NS = info.num_cores, info.num_subcores
    NW = NC * NS
    assert E % NW == 0
    e_per_w = E // NW
    CH = 1000
    assert e_per_w % CH == 0
    n_ch = e_per_w // CH
    mesh = plsc.VectorSubcoreMesh(core_axis_name="c", subcore_axis_name="s")

    @functools.partial(
        pl.kernel,
        mesh=mesh,
        out_type=jax.ShapeDtypeStruct((E, D), jnp.float32),
        compiler_params=pltpu.CompilerParams(use_tc_tiling_on_sc=False),
        scratch_types=[
            pltpu.VMEM((2, CH), jnp.int32),
            pltpu.VMEM((2, CH, D), jnp.float32),
            pltpu.SemaphoreType.DMA,
            pltpu.SemaphoreType.DMA,
        ],
    )
    def k(table_hbm, idx_hbm, out_hbm, idx_v, rows_v, gsem, osem):
        wid = lax.axis_index("s") * NC + lax.axis_index("c")
        base = wid * e_per_w

        gather_h = [None, None]
        out_h = [None, None]
        pltpu.sync_copy(idx_hbm.at[pl.ds(base, CH)], idx_v.at[0])
        gather_h[0] = pltpu.async_copy(table_hbm.at[idx_v.at[0]], rows_v.at[0], gsem)
        for j in range(n_ch):
            b = j % 2
            nb = (j + 1) % 2
            if j + 1 < n_ch:
                noff = base + (j + 1) * CH
                pltpu.sync_copy(idx_hbm.at[pl.ds(noff, CH)], idx_v.at[nb])
                if out_h[nb] is not None:
                    out_h[nb].wait()
                gather_h[nb] = pltpu.async_copy(
                    table_hbm.at[idx_v.at[nb]], rows_v.at[nb], gsem
                )
            gather_h[b].wait()
            out_h[b] = pltpu.async_copy(
                rows_v.at[b], out_hbm.at[pl.ds(base + j * CH, CH)], osem
            )
        out_h[(n_ch - 2) % 2].wait()
        out_h[(n_ch - 1) % 2].wait()

    return k(table, idx)


def _tc_init(na, wnrow, binrow):
    BN = na.shape[0]
    HA = wnrow.shape[1]
    grid = BN // _NPB2

    def body(na_ref, w_ref, b_ref, atom_ref):
        atom_ref[...] = na_ref[...] * w_ref[...] + b_ref[...]

    return pl.pallas_call(
        body,
        grid=(grid,),
        in_specs=[
            pl.BlockSpec((_NPB2, 1), lambda i: (i, 0)),
            pl.BlockSpec((1, HA), lambda i: (0, 0)),
            pl.BlockSpec((1, HA), lambda i: (0, 0)),
        ],
        out_specs=pl.BlockSpec((_NPB2, HA), lambda i: (i, 0)),
        out_shape=jax.ShapeDtypeStruct((BN, HA), jnp.float32),
    )(na, wnrow, binrow)


def _tc_layer(anbr3, ef3, atom, wsF, wsC, bfF, bfC,
              wnFk, wnCk, weF0, weF1, weC0, weC1,
              ghFt, bhFt, ghCt, bhCt, go2, bo2, Kfold, Tfold, S,
              final, extras):
    BN, HA = atom.shape
    grid = BN // _NPB
    Sn = float(BN)

    c0 = lambda k, i: (0, 0)
    edge_map = lambda k, i: (0, jnp.where(k == 0, i, 0), 0)
    node_map = lambda k, i: (i, 0)
    out_map = lambda k, i: (jnp.where(k == 2, i, 0), 0)

    def body(a_ref, e_ref, atom_ref, wsF_ref, wsC_ref, bfF_ref, bfC_ref,
             wnF_ref, wnC_ref, weF0_ref, weF1_ref, weC0_ref, weC1_ref,
             ghF_ref, bhF_ref, ghC_ref, bhC_ref,
             go_ref, bo_ref, K_ref, T_ref, *rest):
        if final:
            we_ref, b0_ref, out_ref, g_s, st_s, sm_s, st2_s = rest
        else:
            natom_ref, g_s, st_s, sm_s, st2_s = rest
        k = pl.program_id(0)
        i = pl.program_id(1)

        def gate_halves(get_a, get_e, wn_dtype):
            at = atom_ref[...]
            sF = (
                jnp.dot(at, wsF_ref[...], preferred_element_type=jnp.float32)
                + bfF_ref[...]
            )
            sC = (
                jnp.dot(at, wsC_ref[...], preferred_element_type=jnp.float32)
                + bfC_ref[...]
            )
            eh = [get_e(0), get_e(1)]
            weF = [weF0_ref[...], weF1_ref[...]]
            weC = [weC0_ref[...], weC1_ref[...]]
            wnF = wnF_ref[...].astype(wn_dtype)
            wnC = wnC_ref[...].astype(wn_dtype)
            outs = []
            for kk in range(4):
                a2 = get_a(kk)
                e2 = eh[kk // 2]
                gF = (
                    jnp.dot(a2, wnF, preferred_element_type=jnp.float32)
                    + jnp.dot(e2, weF[kk % 2], preferred_element_type=jnp.float32)
                    + sF
                )
                gC = (
                    jnp.dot(a2, wnC, preferred_element_type=jnp.float32)
                    + jnp.dot(e2, weC[kk % 2], preferred_element_type=jnp.float32)
                    + sC
                )
                outs.append((gF, gC))
            return outs

        @pl.when(k == 0)
        def _phase_stats():
            s1F = s2F = s1C = s2C = None
            for kk, (gF, gC) in enumerate(gate_halves(
                lambda kk: a_ref[kk], lambda j: e_ref[j], jnp.float32
            )):
                g_s[kk, pl.ds(i * _NPB, _NPB), :] = gF.astype(jnp.bfloat16)
                g_s[4 + kk, pl.ds(i * _NPB, _NPB), :] = gC.astype(jnp.bfloat16)
                p1F = jnp.sum(gF, axis=0, keepdims=True)
                p2F = jnp.sum(gF * gF, axis=0, keepdims=True)
                p1C = jnp.sum(gC, axis=0, keepdims=True)
                p2C = jnp.sum(gC * gC, axis=0, keepdims=True)
                if s1F is None:
                    s1F, s2F, s1C, s2C = p1F, p2F, p1C, p2C
                else:
                    s1F, s2F, s1C, s2C = s1F + p1F, s2F + p2F, s1C + p1C, s2C + p2C
            upd = jnp.concatenate([s1F, s2F, s1C, s2C], axis=0)

            @pl.when(i == 0)
            def _():
                st_s[...] = jnp.zeros_like(st_s)

            st_s[...] += upd

        @pl.when(k == 1)
        def _phase_apply():
            stf = jnp.dot(st_s[...], K_ref[...], preferred_element_type=jnp.float32)
            m1F = stf[0:1, :] / S
            vF = stf[1:2, :] / S - m1F * m1F
            aF = ghF_ref[...] * lax.rsqrt(vF + _EPS)
            cF = bhF_ref[...] - m1F * aF
            m1C = stf[2:3, :] / S
            vC = stf[3:4, :] / S - m1C * m1C
            aC = ghC_ref[...] * lax.rsqrt(vC + _EPS)
            cC = bhC_ref[...] - m1C * aC
            tot = None
            for kk in range(4):
                gF = g_s[kk, pl.ds(i * _NPB, _NPB), :]
                gC = g_s[4 + kk, pl.ds(i * _NPB, _NPB), :]
                p = jax.nn.sigmoid(gF * aF + cF) * jnp.maximum(gC * aC + cC, 0.0)
                tot = p if tot is None else tot + p
            sm = jnp.dot(tot, T_ref[...], preferred_element_type=jnp.float32)
            sm_s[pl.ds(i * _NPB, _NPB), :] = sm
            t1 = jnp.sum(sm, axis=0, keepdims=True)
            t2 = jnp.sum(sm * sm, axis=0, keepdims=True)

            @pl.when(i == 0)
            def _():
                st2_s[...] = jnp.zeros_like(st2_s)

            st2_s[...] += jnp.concatenate([t1, t2], axis=0)

        @pl.when(k == 2)
        def _phase_node():
            st_v = st2_s[...]
            m1 = st_v[0:1, :] / Sn
            v = st_v[1:2, :] / Sn - m1 * m1
            aa = go_ref[...] * lax.rsqrt(v + _EPS)
            cc = bo_ref[...] - m1 * aa
            sm = sm_s[pl.ds(i * _NPB, _NPB), :]
            na_ = jnp.maximum(atom_ref[...] + aa * sm + cc, 0.0)
            if final:
                val = jnp.sum(na_ * we_ref[...])

                @pl.when(i == 0)
                def _():
                    out_ref[...] = Sn * b0_ref[...]

                out_ref[...] += val.reshape(1, 1)
            else:
                natom_ref[...] = na_

    in_specs = [
        pl.BlockSpec((4, _NPB, 128), edge_map),
        pl.BlockSpec((2, _NPB, 128), edge_map),
        pl.BlockSpec((_NPB, HA), node_map),
        pl.BlockSpec((HA, 128), c0),
        pl.BlockSpec((HA, 128), c0),
        pl.BlockSpec((1, 128), c0),
        pl.BlockSpec((1, 128), c0),
        pl.BlockSpec((128, 128), c0),
        pl.BlockSpec((128, 128), c0),
        pl.BlockSpec((128, 128), c0),
        pl.BlockSpec((128, 128), c0),
        pl.BlockSpec((128, 128), c0),
        pl.BlockSpec((128, 128), c0),
        pl.BlockSpec((1, 128), c0),
        pl.BlockSpec((1, 128), c0),
        pl.BlockSpec((1, 128), c0),
        pl.BlockSpec((1, 128), c0),
        pl.BlockSpec((1, HA), c0),
        pl.BlockSpec((1, HA), c0),
        pl.BlockSpec((128, 128), c0),
        pl.BlockSpec((128, HA), c0),
    ]
    args = [anbr3, ef3, atom, wsF, wsC, bfF, bfC,
            wnFk, wnCk, weF0, weF1, weC0, weC1,
            ghFt, bhFt, ghCt, bhCt, go2, bo2, Kfold, Tfold]
    if final:
        werow, b0 = extras
        in_specs += [pl.BlockSpec((1, HA), c0), pl.BlockSpec((1, 1), c0)]
        args += [werow, b0]
        out_specs = pl.BlockSpec((1, 1), c0)
        out_shape = jax.ShapeDtypeStruct((1, 1), jnp.float32)
    else:
        out_specs = pl.BlockSpec((_NPB, HA), out_map)
        out_shape = jax.ShapeDtypeStruct((BN, HA), jnp.float32)

    return pl.pallas_call(
        body,
        grid=(3, grid),
        in_specs=in_specs,
        out_specs=out_specs,
        out_shape=out_shape,
        scratch_shapes=[
            pltpu.VMEM((8, BN, 128), jnp.bfloat16),
            pltpu.VMEM((4, 128), jnp.float32),
            pltpu.VMEM((BN, HA), jnp.float32),
            pltpu.VMEM((2, HA), jnp.float32),
        ],
        compiler_params=pltpu.CompilerParams(
            dimension_semantics=("arbitrary", "arbitrary"),
            vmem_limit_bytes=112 * 1024 * 1024,
        ),
    )(*args)


def kernel(node_attr, edge_attr, edge_idx, Wn, b_in, Wf, bf, gh, bh, go, bo, We, b_out):
    B, N, M = edge_idx.shape
    HA = Wn.shape[0]
    HB = edge_attr.shape[-1]
    BN = B * N
    E = BN * M
    MG = M // 4
    EG = M // 8

    f32 = jnp.float32
    eye4 = jnp.eye(4, dtype=f32)
    eye8 = jnp.eye(8, dtype=f32)

    na = node_attr.reshape(BN, 1)

    idx_off = edge_idx + (jnp.arange(B, dtype=edge_idx.dtype) * N)[:, None, None]
    idx_r = idx_off.reshape(BN, MG, 4).transpose(1, 0, 2).reshape(E)

    ef3 = (
        edge_attr.reshape(BN, EG, 8, HB).transpose(1, 0, 2, 3).reshape(EG, BN, 128)
    ).astype(jnp.bfloat16)

    r128 = np.arange(128)
    Kfold = jnp.asarray((r128[:, None] % HA == r128[None, :] % HA), dtype=f32)
    Tfold = jnp.asarray((r128[:, None] % HA == np.arange(HA)[None, :]), dtype=f32)

    def tile4(x):
        return jnp.concatenate([x] * 4, axis=1)

    n_layers = Wf.shape[0]
    layers = []
    for i in range(n_layers):
        Wfi = Wf[i]
        k8F = jnp.kron(eye8, Wfi[:HA, 2 * HA :].T)
        k8C = jnp.kron(eye8, Wfi[HA:, 2 * HA :].T)
        layers.append(dict(
            wsF=jnp.concatenate([Wfi[:HA, :HA].T] * 4, axis=1),
            wsC=jnp.concatenate([Wfi[HA:, :HA].T] * 4, axis=1),
            wnFk=jnp.kron(eye4, Wfi[:HA, HA : 2 * HA].T),
            wnCk=jnp.kron(eye4, Wfi[HA:, HA : 2 * HA].T),
            weF0=k8F[:, :128].astype(jnp.bfloat16),
            weF1=k8F[:, 128:].astype(jnp.bfloat16),
            weC0=k8C[:, :128].astype(jnp.bfloat16),
            weC1=k8C[:, 128:].astype(jnp.bfloat16),
            bfF=tile4(bf[i][:HA].reshape(1, HA)),
            bfC=tile4(bf[i][HA:].reshape(1, HA)),
            ghFt=tile4(gh[i][:HA].reshape(1, HA)),
            ghCt=tile4(gh[i][HA:].reshape(1, HA)),
            bhFt=tile4(bh[i][:HA].reshape(1, HA)),
            bhCt=tile4(bh[i][HA:].reshape(1, HA)),
            go2=go[i].reshape(1, HA),
            bo2=bo[i].reshape(1, HA),
        ))

    atom = _tc_init(na, Wn.reshape(1, HA), b_in.reshape(1, HA))

    out = None
    for i in range(n_layers):
        Li = layers[i]
        anbr3 = _sc_gather(atom, idx_r).reshape(MG, BN, 128)
        last = i == n_layers - 1
        extras = (We.reshape(1, HA), b_out.reshape(1, 1)) if last else None
        res = _tc_layer(
            anbr3, ef3, atom,
            Li["wsF"], Li["wsC"], Li["bfF"], Li["bfC"],
            Li["wnFk"], Li["wnCk"],
            Li["weF0"], Li["weF1"], Li["weC0"], Li["weC1"],
            Li["ghFt"], Li["bhFt"], Li["ghCt"], Li["bhCt"],
            Li["go2"], Li["bo2"], Kfold, Tfold, float(E),
            last, extras,
        )
        if last:
            out = res
        else:
            atom = res

    return out.reshape(())
